# Initial kernel scaffold; baseline (speedup 1.0000x reference)
import jax
import jax.numpy as jnp
from jax.experimental import pallas as pl


def kernel(x, cluster_centers, temperature, cluster_weights, W1, b1, W2, b2):
    raise NotImplementedError("write your pallas kernel here")


# speedup vs baseline: 11.6497x; 11.6497x over previous
import jax
import jax.numpy as jnp
from jax import lax
from jax.experimental import pallas as pl
from jax.experimental.pallas import tpu as pltpu

_ROWS = 512
_K = 5


def _fused_kernel(x_ref, xT_ref, c_ref, cT_ref, w_ref, temp_ref,
                  W1_ref, b1_ref, W2_ref, b2_ref,
                  enc_ref, assign_ref, knn_ref, stats_ref,
                  intra_ref, inter_ref):
    i = pl.program_id(0)
    nb = pl.num_programs(0)
    R, D = x_ref.shape
    B = xT_ref.shape[1]
    C = c_ref.shape[0]

    xb = x_ref[...]
    xT = xT_ref[...]

    xb_n2 = jnp.sum(xb * xb, axis=1, keepdims=True)
    all_n2 = jnp.sum(xT * xT, axis=0, keepdims=True)

    dot = jnp.dot(xb, xT, preferred_element_type=jnp.float32,
                  precision=lax.Precision.HIGHEST)
    d2 = xb_n2 + all_n2 - 2.0 * dot

    row_g = lax.broadcasted_iota(jnp.int32, (R, B), 0) + i * R
    col_g = lax.broadcasted_iota(jnp.int32, (R, B), 1)
    d2 = jnp.where(row_g == col_g, jnp.inf, d2)

    cT = cT_ref[...]
    c_n2 = jnp.sum(cT * cT, axis=0, keepdims=True)
    dotc = jnp.dot(xb, cT, preferred_element_type=jnp.float32,
                   precision=lax.Precision.HIGHEST)
    d2c = xb_n2 + c_n2 - 2.0 * dotc
    dist_c = jnp.sqrt(jnp.maximum(d2c, 1e-12))

    t = temp_ref[0, 0]
    logits = -dist_c / t
    m = jnp.max(logits, axis=1, keepdims=True)
    e = jnp.exp(logits - m)
    s = jnp.sum(e, axis=1, keepdims=True)
    assign = (e / s) * w_ref[...]
    assign_ref[...] = assign

    pre = jnp.dot(assign, W1_ref[0:C, :], preferred_element_type=jnp.float32,
                  precision=lax.Precision.HIGHEST)

    d2m = d2
    for k in range(_K):
        v = jnp.min(d2m, axis=1, keepdims=True)
        d2m = jnp.where(d2m <= v, jnp.inf, d2m)
        dk = jnp.sqrt(jnp.maximum(v, 1e-12))
        knn_ref[:, k:k + 1] = dk
        pre += dk * W1_ref[C + k:C + k + 1, :]

    lm = jnp.mean(xb, axis=1, keepdims=True)
    ls = jnp.sqrt(jnp.sum((xb - lm) ** 2, axis=1, keepdims=True)
                  / (D - 1)) + 1e-8
    mx = jnp.max(xb, axis=1, keepdims=True)
    ex = jnp.exp(xb - mx)
    sx = jnp.sum(ex, axis=1, keepdims=True)
    logp = xb - mx - jnp.log(sx)
    ent = -jnp.sum((ex / sx) * logp, axis=1, keepdims=True)
    stats_ref[:, 0:1] = lm
    stats_ref[:, 1:2] = ls
    stats_ref[:, 2:3] = ent
    pre += lm * W1_ref[C + _K:C + _K + 1, :]
    pre += ls * W1_ref[C + _K + 1:C + _K + 2, :]
    pre += ent * W1_ref[C + _K + 2:C + _K + 3, :]

    h = jnp.maximum(pre + b1_ref[...], 0.0)
    enc = jnp.dot(h, W2_ref[...], preferred_element_type=jnp.float32,
                  precision=lax.Precision.HIGHEST) + b2_ref[...]
    enc_ref[...] = enc

    @pl.when(i == 0)
    def _init():
        intra_ref[0, 0] = 0.0
        cc = c_ref[...]
        ccn = jnp.sum(cc * cc, axis=1, keepdims=True)
        d2cc = ccn + c_n2 - 2.0 * jnp.dot(
            cc, cT, preferred_element_type=jnp.float32,
            precision=lax.Precision.HIGHEST)
        dcc = jnp.sqrt(jnp.maximum(d2cc, 1e-12))
        ri = lax.broadcasted_iota(jnp.int32, (C, C), 0)
        ci = lax.broadcasted_iota(jnp.int32, (C, C), 1)
        inter_ref[0, 0] = jnp.sum(jnp.where(ri == ci, 0.0, dcc)) / (C * (C - 1))

    intra_ref[0, 0] += jnp.sum(dist_c * assign)

    @pl.when(i == nb - 1)
    def _final():
        intra_ref[0, 0] = intra_ref[0, 0] / (B * C)


def kernel(x, cluster_centers, temperature, cluster_weights, W1, b1, W2, b2):
    B, D = x.shape
    C = cluster_centers.shape[0]
    H = W1.shape[1]
    O = W2.shape[1]
    R = _ROWS
    nb = B // R

    xT = x.T
    cT = cluster_centers.T
    w_row = cluster_weights.reshape(1, C)
    temp = temperature.reshape(1, 1)
    b1r = b1.reshape(1, H)
    b2r = b2.reshape(1, O)

    f32 = jnp.float32
    out_shape = [
        jax.ShapeDtypeStruct((B, O), f32),
        jax.ShapeDtypeStruct((B, C), f32),
        jax.ShapeDtypeStruct((B, _K), f32),
        jax.ShapeDtypeStruct((B, 3), f32),
        jax.ShapeDtypeStruct((1, 1), f32),
        jax.ShapeDtypeStruct((1, 1), f32),
    ]
    smem = pltpu.SMEM
    in_specs = [
        pl.BlockSpec((R, D), lambda i: (i, 0)),
        pl.BlockSpec((D, B), lambda i: (0, 0)),
        pl.BlockSpec((C, D), lambda i: (0, 0)),
        pl.BlockSpec((D, C), lambda i: (0, 0)),
        pl.BlockSpec((1, C), lambda i: (0, 0)),
        pl.BlockSpec(memory_space=smem),
        pl.BlockSpec((C + _K + 3, H), lambda i: (0, 0)),
        pl.BlockSpec((1, H), lambda i: (0, 0)),
        pl.BlockSpec((H, O), lambda i: (0, 0)),
        pl.BlockSpec((1, O), lambda i: (0, 0)),
    ]
    out_specs = [
        pl.BlockSpec((R, O), lambda i: (i, 0)),
        pl.BlockSpec((R, C), lambda i: (i, 0)),
        pl.BlockSpec((R, _K), lambda i: (i, 0)),
        pl.BlockSpec((R, 3), lambda i: (i, 0)),
        pl.BlockSpec(memory_space=smem),
        pl.BlockSpec(memory_space=smem),
    ]
    enc, assign, knn_d, stats, intra, inter = pl.pallas_call(
        _fused_kernel,
        grid=(nb,),
        in_specs=in_specs,
        out_specs=out_specs,
        out_shape=out_shape,
        compiler_params=pltpu.CompilerParams(
            dimension_semantics=("arbitrary",)),
    )(x, xT, cluster_centers, cT, w_row, temp, W1, b1r, W2, b2r)

    intra_s = intra[0, 0]
    inter_s = inter[0, 0]
    loss = intra_s - 0.1 * inter_s
    return (enc, assign, knn_d, stats, loss, intra_s, inter_s)


# speedup vs baseline: 17.5309x; 1.5048x over previous
import jax
import jax.numpy as jnp
from jax import lax
from jax.experimental import pallas as pl
from jax.experimental.pallas import tpu as pltpu

_ROWS = 512
_K = 5


def _fused_kernel(x_ref, xT_ref, c_ref, cT_ref, w_ref, temp_ref,
                  W1_ref, b1_ref, W2_ref, b2_ref,
                  enc_ref, assign_ref, knn_ref, stats_ref,
                  intra_ref, inter_ref):
    i = pl.program_id(0)
    nb = pl.num_programs(0)
    R, D = x_ref.shape
    B = xT_ref.shape[1]
    C = c_ref.shape[0]

    xb = x_ref[...]
    xT = xT_ref[...]

    xb_n2 = jnp.sum(xb * xb, axis=1, keepdims=True)
    all_n2 = jnp.sum(xT * xT, axis=0, keepdims=True)

    dot = jnp.dot(xb, xT, preferred_element_type=jnp.float32)
    d2 = xb_n2 + all_n2 - 2.0 * dot

    row_g = lax.broadcasted_iota(jnp.int32, (R, B), 0) + i * R
    col_g = lax.broadcasted_iota(jnp.int32, (R, B), 1)
    d2 = jnp.where(row_g == col_g, jnp.inf, d2)

    cT = cT_ref[...]
    c_n2 = jnp.sum(cT * cT, axis=0, keepdims=True)
    dotc = jnp.dot(xb, cT, preferred_element_type=jnp.float32)
    d2c = xb_n2 + c_n2 - 2.0 * dotc
    dist_c = jnp.sqrt(jnp.maximum(d2c, 1e-12))

    t = temp_ref[0, 0]
    logits = -dist_c / t
    m = jnp.max(logits, axis=1, keepdims=True)
    e = jnp.exp(logits - m)
    s = jnp.sum(e, axis=1, keepdims=True)
    assign = (e / s) * w_ref[...]
    assign_ref[...] = assign

    pre = jnp.dot(assign, W1_ref[0:C, :], preferred_element_type=jnp.float32)

    d2m = d2
    for k in range(_K):
        v = jnp.min(d2m, axis=1, keepdims=True)
        d2m = jnp.where(d2m <= v, jnp.inf, d2m)
        dk = jnp.sqrt(jnp.maximum(v, 1e-12))
        knn_ref[:, k:k + 1] = dk
        pre += dk * W1_ref[C + k:C + k + 1, :]

    lm = jnp.mean(xb, axis=1, keepdims=True)
    ls = jnp.sqrt(jnp.sum((xb - lm) ** 2, axis=1, keepdims=True)
                  / (D - 1)) + 1e-8
    mx = jnp.max(xb, axis=1, keepdims=True)
    ex = jnp.exp(xb - mx)
    sx = jnp.sum(ex, axis=1, keepdims=True)
    logp = xb - mx - jnp.log(sx)
    ent = -jnp.sum((ex / sx) * logp, axis=1, keepdims=True)
    stats_ref[:, 0:1] = lm
    stats_ref[:, 1:2] = ls
    stats_ref[:, 2:3] = ent
    pre += lm * W1_ref[C + _K:C + _K + 1, :]
    pre += ls * W1_ref[C + _K + 1:C + _K + 2, :]
    pre += ent * W1_ref[C + _K + 2:C + _K + 3, :]

    h = jnp.maximum(pre + b1_ref[...], 0.0)
    enc = jnp.dot(h, W2_ref[...], preferred_element_type=jnp.float32) + b2_ref[...]
    enc_ref[...] = enc

    @pl.when(i == 0)
    def _init():
        intra_ref[0, 0] = 0.0
        cc = c_ref[...]
        ccn = jnp.sum(cc * cc, axis=1, keepdims=True)
        d2cc = ccn + c_n2 - 2.0 * jnp.dot(
            cc, cT, preferred_element_type=jnp.float32)
        dcc = jnp.sqrt(jnp.maximum(d2cc, 1e-12))
        ri = lax.broadcasted_iota(jnp.int32, (C, C), 0)
        ci = lax.broadcasted_iota(jnp.int32, (C, C), 1)
        inter_ref[0, 0] = jnp.sum(jnp.where(ri == ci, 0.0, dcc)) / (C * (C - 1))

    intra_ref[0, 0] += jnp.sum(dist_c * assign)

    @pl.when(i == nb - 1)
    def _final():
        intra_ref[0, 0] = intra_ref[0, 0] / (B * C)


def kernel(x, cluster_centers, temperature, cluster_weights, W1, b1, W2, b2):
    B, D = x.shape
    C = cluster_centers.shape[0]
    H = W1.shape[1]
    O = W2.shape[1]
    R = _ROWS
    nb = B // R

    xT = x.T
    cT = cluster_centers.T
    w_row = cluster_weights.reshape(1, C)
    temp = temperature.reshape(1, 1)
    b1r = b1.reshape(1, H)
    b2r = b2.reshape(1, O)

    f32 = jnp.float32
    out_shape = [
        jax.ShapeDtypeStruct((B, O), f32),
        jax.ShapeDtypeStruct((B, C), f32),
        jax.ShapeDtypeStruct((B, _K), f32),
        jax.ShapeDtypeStruct((B, 3), f32),
        jax.ShapeDtypeStruct((1, 1), f32),
        jax.ShapeDtypeStruct((1, 1), f32),
    ]
    smem = pltpu.SMEM
    in_specs = [
        pl.BlockSpec((R, D), lambda i: (i, 0)),
        pl.BlockSpec((D, B), lambda i: (0, 0)),
        pl.BlockSpec((C, D), lambda i: (0, 0)),
        pl.BlockSpec((D, C), lambda i: (0, 0)),
        pl.BlockSpec((1, C), lambda i: (0, 0)),
        pl.BlockSpec(memory_space=smem),
        pl.BlockSpec((C + _K + 3, H), lambda i: (0, 0)),
        pl.BlockSpec((1, H), lambda i: (0, 0)),
        pl.BlockSpec((H, O), lambda i: (0, 0)),
        pl.BlockSpec((1, O), lambda i: (0, 0)),
    ]
    out_specs = [
        pl.BlockSpec((R, O), lambda i: (i, 0)),
        pl.BlockSpec((R, C), lambda i: (i, 0)),
        pl.BlockSpec((R, _K), lambda i: (i, 0)),
        pl.BlockSpec((R, 3), lambda i: (i, 0)),
        pl.BlockSpec(memory_space=smem),
        pl.BlockSpec(memory_space=smem),
    ]
    enc, assign, knn_d, stats, intra, inter = pl.pallas_call(
        _fused_kernel,
        grid=(nb,),
        in_specs=in_specs,
        out_specs=out_specs,
        out_shape=out_shape,
        compiler_params=pltpu.CompilerParams(
            dimension_semantics=("arbitrary",)),
    )(x, xT, cluster_centers, cT, w_row, temp, W1, b1r, W2, b2r)

    intra_s = intra[0, 0]
    inter_s = inter[0, 0]
    loss = intra_s - 0.1 * inter_s
    return (enc, assign, knn_d, stats, loss, intra_s, inter_s)
